# Initial kernel scaffold; baseline (speedup 1.0000x reference)
import functools
import math

import jax
import jax.numpy as jnp
import numpy as np
from jax.experimental import pallas as pl
from jax.experimental.pallas import tpu as pltpu

_NUM_MASKS = 100
_MIN_CLUSTER = 3
_MAX_CLUSTER = 6


@functools.cache
def _mask_params(B, H, W):
    kc, kp = jax.random.split(jax.random.key(42))
    cs = int(jax.random.randint(kc, (), _MIN_CLUSTER, _MAX_CLUSTER))
    mh, mw = min(H, cs), min(W, cs)
    S = math.ceil(_NUM_MASKS / (mh * mw))
    keys = jax.random.split(kp, B)
    idx = np.stack(
        [np.asarray(jax.random.permutation(keys[b], H * W))[:S] for b in range(B)]
    )
    return idx.astype(np.int32), mh, mw, S


def _masker_kernel(idx_ref, x_ref, emb_ref, out_ref, mout_ref, *, H, W, mh, mw, S):
    b = pl.program_id(0)
    HW = H * W
    p = jax.lax.broadcasted_iota(jnp.int32, (1, HW), 1)
    ii = p // W
    jj = p % W
    fh = (mh - 1) // 2
    fw = (mw - 1) // 2
    m = None
    for s in range(S):
        q = idx_ref[b, s]
        qi = q // W
        qj = q % W
        c = ((ii >= qi - fh) & (ii <= qi + (mh - 1 - fh))
             & (jj >= qj - fw) & (jj <= qj + (mw - 1 - fw)))
        m = c if m is None else (m | c)
    x = x_ref[...]
    emb = emb_ref[...]
    out_ref[...] = jnp.where(m.reshape(1, 1, HW), emb, x)
    mout_ref[...] = m.reshape(1, 1, HW).astype(jnp.int32)


def kernel(input, mask_embedding):
    B, C, H, W = input.shape
    idx, mh, mw, S = _mask_params(B, H, W)
    HW = H * W
    x = input.reshape(B, C, HW)
    emb = mask_embedding.reshape(1, C, 1)
    out, mout = pl.pallas_call(
        functools.partial(_masker_kernel, H=H, W=W, mh=mh, mw=mw, S=S),
        grid=(B,),
        in_specs=[
            pl.BlockSpec(memory_space=pltpu.SMEM),
            pl.BlockSpec((1, C, HW), lambda b: (b, 0, 0)),
            pl.BlockSpec((1, C, 1), lambda b: (0, 0, 0)),
        ],
        out_specs=[
            pl.BlockSpec((1, C, HW), lambda b: (b, 0, 0)),
            pl.BlockSpec((1, 1, HW), lambda b: (b, 0, 0)),
        ],
        out_shape=[
            jax.ShapeDtypeStruct((B, C, HW), input.dtype),
            jax.ShapeDtypeStruct((B, 1, HW), jnp.int32),
        ],
    )(jnp.asarray(idx), x, emb)
    return out.reshape(B, C, H, W), (mout.reshape(B, H, W) > 0)


# speedup vs baseline: 2.5697x; 2.5697x over previous
import functools
import math

import jax
import jax.numpy as jnp
import numpy as np
from jax.experimental import pallas as pl
from jax.experimental.pallas import tpu as pltpu

_NUM_MASKS = 100
_MIN_CLUSTER = 3
_MAX_CLUSTER = 6


@functools.cache
def _mask_params(B, H, W):
    with jax.ensure_compile_time_eval():
        kc, kp = jax.random.split(jax.random.key(42))
        cs = int(jax.random.randint(kc, (), _MIN_CLUSTER, _MAX_CLUSTER))
        mh, mw = min(H, cs), min(W, cs)
        S = math.ceil(_NUM_MASKS / (mh * mw))
        keys = jax.random.split(kp, B)
        idx = np.stack(
            [np.asarray(jax.random.permutation(keys[b], H * W))[:S] for b in range(B)]
        )
    return idx.astype(np.int32), mh, mw, S


def _masker_kernel(idx_ref, x_ref, emb_ref, out_ref, mout_ref, *, H, W, mh, mw, S):
    b = pl.program_id(0)
    HW = H * W
    p = jax.lax.broadcasted_iota(jnp.int32, (1, HW), 1)
    ii = p // W
    jj = p % W
    fh = (mh - 1) // 2
    fw = (mw - 1) // 2
    m = None
    for s in range(S):
        q = idx_ref[b, s]
        qi = q // W
        qj = q % W
        c = ((ii >= qi - fh) & (ii <= qi + (mh - 1 - fh))
             & (jj >= qj - fw) & (jj <= qj + (mw - 1 - fw)))
        m = c if m is None else (m | c)
    x = x_ref[...]
    emb = emb_ref[...]
    out_ref[...] = jnp.where(m.reshape(1, 1, HW), emb, x)
    mout_ref[...] = m.reshape(1, 1, HW).astype(jnp.int32)


def kernel(input, mask_embedding):
    B, C, H, W = input.shape
    idx, mh, mw, S = _mask_params(B, H, W)
    HW = H * W
    x = input.reshape(B, C, HW)
    emb = mask_embedding.reshape(1, C, 1)
    out, mout = pl.pallas_call(
        functools.partial(_masker_kernel, H=H, W=W, mh=mh, mw=mw, S=S),
        grid=(B,),
        in_specs=[
            pl.BlockSpec(memory_space=pltpu.SMEM),
            pl.BlockSpec((1, C, HW), lambda b: (b, 0, 0)),
            pl.BlockSpec((1, C, 1), lambda b: (0, 0, 0)),
        ],
        out_specs=[
            pl.BlockSpec((1, C, HW), lambda b: (b, 0, 0)),
            pl.BlockSpec((1, 1, HW), lambda b: (b, 0, 0)),
        ],
        out_shape=[
            jax.ShapeDtypeStruct((B, C, HW), input.dtype),
            jax.ShapeDtypeStruct((B, 1, HW), jnp.int32),
        ],
    )(jnp.asarray(idx), x, emb)
    return out.reshape(B, C, H, W), (mout.reshape(B, H, W) > 0)


# speedup vs baseline: 8.4471x; 3.2872x over previous
import functools
import math

import jax
import jax.numpy as jnp
import numpy as np
from jax.experimental import pallas as pl
from jax.experimental.pallas import tpu as pltpu

_NUM_MASKS = 100
_MIN_CLUSTER = 3
_MAX_CLUSTER = 6


@functools.cache
def _mask_params(B, H, W):
    with jax.ensure_compile_time_eval():
        kc, kp = jax.random.split(jax.random.key(42))
        cs = int(jax.random.randint(kc, (), _MIN_CLUSTER, _MAX_CLUSTER))
        mh, mw = min(H, cs), min(W, cs)
        S = math.ceil(_NUM_MASKS / (mh * mw))
        keys = jax.random.split(kp, B)
        idx = np.stack(
            [np.asarray(jax.random.permutation(keys[b], H * W))[:S] for b in range(B)]
        ).astype(np.int32)
    fh, fw = (mh - 1) // 2, (mw - 1) // 2
    ii = np.arange(H)[:, None]
    jj = np.arange(W)[None, :]
    mask = np.zeros((B, H, W), np.bool_)
    for b in range(B):
        for q in idx[b]:
            qi, qj = q // W, q % W
            mask[b] |= ((ii >= qi - fh) & (ii <= qi + mh - 1 - fh)
                        & (jj >= qj - fw) & (jj <= qj + mw - 1 - fw))
    mcol = mask.reshape(B, H * W, 1).astype(np.float32)
    return idx, mcol, mh, mw, S


def _masker_kernel(idx_ref, x_ref, mcol_ref, emb_ref, out_ref, mout_ref,
                   *, H, W, mh, mw, S):
    b = pl.program_id(0)
    HW = H * W
    sel = mcol_ref[...] != 0.0
    out_ref[...] = jnp.where(sel, emb_ref[...], x_ref[...])
    p = jax.lax.broadcasted_iota(jnp.int32, (1, 1, HW), 2)
    ii = p // W
    jj = p % W
    fh = (mh - 1) // 2
    fw = (mw - 1) // 2
    m = None
    for s in range(S):
        q = idx_ref[b, s]
        qi = q // W
        qj = q % W
        c = ((ii >= qi - fh) & (ii <= qi + (mh - 1 - fh))
             & (jj >= qj - fw) & (jj <= qj + (mw - 1 - fw)))
        m = c if m is None else (m | c)
    mout_ref[...] = m.astype(jnp.int32)


def kernel(input, mask_embedding):
    B, C, H, W = input.shape
    idx, mcol, mh, mw, S = _mask_params(B, H, W)
    HW = H * W
    x = jnp.transpose(input, (0, 2, 3, 1)).reshape(B, HW, C)
    emb = mask_embedding.reshape(1, 1, C)
    out, mout = pl.pallas_call(
        functools.partial(_masker_kernel, H=H, W=W, mh=mh, mw=mw, S=S),
        grid=(B,),
        in_specs=[
            pl.BlockSpec(memory_space=pltpu.SMEM),
            pl.BlockSpec((1, HW, C), lambda b: (b, 0, 0)),
            pl.BlockSpec((1, HW, 1), lambda b: (b, 0, 0)),
            pl.BlockSpec((1, 1, C), lambda b: (0, 0, 0)),
        ],
        out_specs=[
            pl.BlockSpec((1, HW, C), lambda b: (b, 0, 0)),
            pl.BlockSpec((1, 1, HW), lambda b: (b, 0, 0)),
        ],
        out_shape=[
            jax.ShapeDtypeStruct((B, HW, C), input.dtype),
            jax.ShapeDtypeStruct((B, 1, HW), jnp.int32),
        ],
    )(jnp.asarray(idx), x, jnp.asarray(mcol), emb)
    out4 = jnp.transpose(out.reshape(B, H, W, C), (0, 3, 1, 2))
    return out4, (mout.reshape(B, H, W) > 0)


# speedup vs baseline: 8.7250x; 1.0329x over previous
import functools
import math

import jax
import jax.numpy as jnp
import numpy as np
from jax.experimental import pallas as pl
from jax.experimental.pallas import tpu as pltpu

_NUM_MASKS = 100
_MIN_CLUSTER = 3
_MAX_CLUSTER = 6


@functools.cache
def _mask_params(B, H, W):
    with jax.ensure_compile_time_eval():
        kc, kp = jax.random.split(jax.random.key(42))
        cs = int(jax.random.randint(kc, (), _MIN_CLUSTER, _MAX_CLUSTER))
        mh, mw = min(H, cs), min(W, cs)
        S = math.ceil(_NUM_MASKS / (mh * mw))
        keys = jax.random.split(kp, B)
        idx = np.stack(
            [np.asarray(jax.random.permutation(keys[b], H * W))[:S] for b in range(B)]
        ).astype(np.int32)
    fh, fw = (mh - 1) // 2, (mw - 1) // 2
    ii = np.arange(H)[:, None]
    jj = np.arange(W)[None, :]
    mask = np.zeros((B, H, W), np.bool_)
    for b in range(B):
        for q in idx[b]:
            qi, qj = q // W, q % W
            mask[b] |= ((ii >= qi - fh) & (ii <= qi + mh - 1 - fh)
                        & (jj >= qj - fw) & (jj <= qj + mw - 1 - fw))
    mcol = mask.reshape(B, H * W, 1).astype(np.int8)
    return idx, mcol, mh, mw, S


def _masker_kernel(idx_ref, x_ref, mcol_ref, emb_ref, out_ref, mout_ref,
                   *, H, W, mh, mw, S):
    b = pl.program_id(0)
    HW = H * W
    sel = mcol_ref[...] != 0
    out_ref[...] = jnp.where(sel, emb_ref[...], x_ref[...])
    p = jax.lax.broadcasted_iota(jnp.int32, (1, 1, HW), 2)
    ii = p // W
    jj = p % W
    fh = (mh - 1) // 2
    fw = (mw - 1) // 2
    m = None
    for s in range(S):
        q = idx_ref[b, s]
        qi = q // W
        qj = q % W
        c = ((ii >= qi - fh) & (ii <= qi + (mh - 1 - fh))
             & (jj >= qj - fw) & (jj <= qj + (mw - 1 - fw)))
        m = c if m is None else (m | c)
    mout_ref[...] = m.astype(jnp.int32)


def kernel(input, mask_embedding):
    B, C, H, W = input.shape
    idx, mcol, mh, mw, S = _mask_params(B, H, W)
    HW = H * W
    x = jnp.transpose(input, (0, 2, 3, 1)).reshape(B, HW, C)
    emb = mask_embedding.reshape(1, 1, C)
    out, mout = pl.pallas_call(
        functools.partial(_masker_kernel, H=H, W=W, mh=mh, mw=mw, S=S),
        grid=(B,),
        in_specs=[
            pl.BlockSpec(memory_space=pltpu.SMEM),
            pl.BlockSpec((1, HW, C), lambda b: (b, 0, 0)),
            pl.BlockSpec((1, HW, 1), lambda b: (b, 0, 0)),
            pl.BlockSpec((1, 1, C), lambda b: (0, 0, 0)),
        ],
        out_specs=[
            pl.BlockSpec((1, HW, C), lambda b: (b, 0, 0)),
            pl.BlockSpec((1, 1, HW), lambda b: (b, 0, 0)),
        ],
        out_shape=[
            jax.ShapeDtypeStruct((B, HW, C), input.dtype),
            jax.ShapeDtypeStruct((B, 1, HW), jnp.int32),
        ],
    )(jnp.asarray(idx), x, jnp.asarray(mcol), emb)
    out4 = jnp.transpose(out.reshape(B, H, W, C), (0, 3, 1, 2))
    return out4, (mout.reshape(B, H, W) > 0)


# speedup vs baseline: 9.1355x; 1.0470x over previous
import functools
import math

import jax
import jax.numpy as jnp
import numpy as np
from jax.experimental import pallas as pl
from jax.experimental.pallas import tpu as pltpu

_NUM_MASKS = 100
_MIN_CLUSTER = 3
_MAX_CLUSTER = 6


@functools.cache
def _mask_params(B, H, W):
    with jax.ensure_compile_time_eval():
        kc, kp = jax.random.split(jax.random.key(42))
        cs = int(jax.random.randint(kc, (), _MIN_CLUSTER, _MAX_CLUSTER))
        mh, mw = min(H, cs), min(W, cs)
        S = math.ceil(_NUM_MASKS / (mh * mw))
        keys = jax.random.split(kp, B)
        idx = np.stack(
            [np.asarray(jax.random.permutation(keys[b], H * W))[:S] for b in range(B)]
        ).astype(np.int32)
    fh, fw = (mh - 1) // 2, (mw - 1) // 2
    ii = np.arange(H)[:, None]
    jj = np.arange(W)[None, :]
    mask = np.zeros((B, H, W), np.bool_)
    for b in range(B):
        for q in idx[b]:
            qi, qj = q // W, q % W
            mask[b] |= ((ii >= qi - fh) & (ii <= qi + mh - 1 - fh)
                        & (jj >= qj - fw) & (jj <= qj + mw - 1 - fw))
    mcol = mask.reshape(B, H * W, 1).astype(np.int8)
    return idx, mcol, mh, mw, S


def _masker_kernel(idx_ref, x_ref, mcol_ref, emb_ref, out_ref, mout_ref,
                   *, H, W, mh, mw, S, BB):
    g = pl.program_id(0)
    HW = H * W
    sel = mcol_ref[...] != 0
    out_ref[...] = jnp.where(sel, emb_ref[...], x_ref[...])
    p = jax.lax.broadcasted_iota(jnp.int32, (1, 1, HW), 2)
    ii = p // W
    jj = p % W
    fh = (mh - 1) // 2
    fw = (mw - 1) // 2
    for k in range(BB):
        m = None
        for s in range(S):
            q = idx_ref[g * BB + k, s]
            qi = q // W
            qj = q % W
            c = ((ii >= qi - fh) & (ii <= qi + (mh - 1 - fh))
                 & (jj >= qj - fw) & (jj <= qj + (mw - 1 - fw)))
            m = c if m is None else (m | c)
        mout_ref[k] = m.astype(jnp.int32)[0]


def kernel(input, mask_embedding):
    B, C, H, W = input.shape
    idx, mcol, mh, mw, S = _mask_params(B, H, W)
    HW = H * W
    x = jnp.transpose(input, (0, 2, 3, 1)).reshape(B, HW, C)
    emb = mask_embedding.reshape(1, 1, C)
    BB = 2 if B % 2 == 0 else 1
    out, mout = pl.pallas_call(
        functools.partial(_masker_kernel, H=H, W=W, mh=mh, mw=mw, S=S, BB=BB),
        grid=(B // BB,),
        in_specs=[
            pl.BlockSpec(memory_space=pltpu.SMEM),
            pl.BlockSpec((BB, HW, C), lambda b: (b, 0, 0)),
            pl.BlockSpec((BB, HW, 1), lambda b: (b, 0, 0)),
            pl.BlockSpec((1, 1, C), lambda b: (0, 0, 0)),
        ],
        out_specs=[
            pl.BlockSpec((BB, HW, C), lambda b: (b, 0, 0)),
            pl.BlockSpec((BB, 1, HW), lambda b: (b, 0, 0)),
        ],
        out_shape=[
            jax.ShapeDtypeStruct((B, HW, C), input.dtype),
            jax.ShapeDtypeStruct((B, 1, HW), jnp.int32),
        ],
    )(jnp.asarray(idx), x, jnp.asarray(mcol), emb)
    out4 = jnp.transpose(out.reshape(B, H, W, C), (0, 3, 1, 2))
    return out4, (mout.reshape(B, H, W) > 0)


# speedup vs baseline: 9.1661x; 1.0034x over previous
import functools
import math

import jax
import jax.numpy as jnp
import numpy as np
from jax.experimental import pallas as pl
from jax.experimental.pallas import tpu as pltpu

_NUM_MASKS = 100
_MIN_CLUSTER = 3
_MAX_CLUSTER = 6


@functools.cache
def _mask_params(B, H, W):
    with jax.ensure_compile_time_eval():
        kc, kp = jax.random.split(jax.random.key(42))
        cs = int(jax.random.randint(kc, (), _MIN_CLUSTER, _MAX_CLUSTER))
        mh, mw = min(H, cs), min(W, cs)
        S = math.ceil(_NUM_MASKS / (mh * mw))
        keys = jax.random.split(kp, B)
        idx = np.stack(
            [np.asarray(jax.random.permutation(keys[b], H * W))[:S] for b in range(B)]
        ).astype(np.int32)
    fh, fw = (mh - 1) // 2, (mw - 1) // 2
    ii = np.arange(H)[:, None]
    jj = np.arange(W)[None, :]
    mask = np.zeros((B, H, W), np.bool_)
    for b in range(B):
        for q in idx[b]:
            qi, qj = q // W, q % W
            mask[b] |= ((ii >= qi - fh) & (ii <= qi + mh - 1 - fh)
                        & (jj >= qj - fw) & (jj <= qj + mw - 1 - fw))
    mcol = mask.reshape(B, H * W, 1).astype(np.int8)
    return idx, mcol, mh, mw, S


def _masker_kernel(idx_ref, x_ref, mcol_ref, emb_ref, out_ref, mout_ref,
                   *, H, W, mh, mw, S, BB):
    g = pl.program_id(0)
    HW = H * W
    sel = mcol_ref[...] != 0
    out_ref[...] = jnp.where(sel, emb_ref[...], x_ref[...])
    p = jax.lax.broadcasted_iota(jnp.int32, (1, 1, HW), 2)
    ii = p // W
    jj = p % W
    fh = (mh - 1) // 2
    fw = (mw - 1) // 2
    for k in range(BB):
        m = None
        for s in range(S):
            q = idx_ref[g * BB + k, s]
            qi = q // W
            qj = q % W
            c = ((ii >= qi - fh) & (ii <= qi + (mh - 1 - fh))
                 & (jj >= qj - fw) & (jj <= qj + (mw - 1 - fw)))
            m = c if m is None else (m | c)
        mout_ref[k] = m.astype(jnp.int32)[0]


def kernel(input, mask_embedding):
    B, C, H, W = input.shape
    idx, mcol, mh, mw, S = _mask_params(B, H, W)
    HW = H * W
    x = jnp.transpose(input, (0, 2, 3, 1)).reshape(B, HW, C)
    emb = mask_embedding.reshape(1, 1, C)
    BB = 4 if B % 4 == 0 else (2 if B % 2 == 0 else 1)
    out, mout = pl.pallas_call(
        functools.partial(_masker_kernel, H=H, W=W, mh=mh, mw=mw, S=S, BB=BB),
        grid=(B // BB,),
        in_specs=[
            pl.BlockSpec(memory_space=pltpu.SMEM),
            pl.BlockSpec((BB, HW, C), lambda b: (b, 0, 0)),
            pl.BlockSpec((BB, HW, 1), lambda b: (b, 0, 0)),
            pl.BlockSpec((1, 1, C), lambda b: (0, 0, 0)),
        ],
        out_specs=[
            pl.BlockSpec((BB, HW, C), lambda b: (b, 0, 0)),
            pl.BlockSpec((BB, 1, HW), lambda b: (b, 0, 0)),
        ],
        out_shape=[
            jax.ShapeDtypeStruct((B, HW, C), input.dtype),
            jax.ShapeDtypeStruct((B, 1, HW), jnp.int32),
        ],
    )(jnp.asarray(idx), x, jnp.asarray(mcol), emb)
    out4 = jnp.transpose(out.reshape(B, H, W, C), (0, 3, 1, 2))
    return out4, (mout.reshape(B, H, W) > 0)
